# baseline (speedup 1.0000x reference)
import functools

import jax
import jax.numpy as jnp
from jax import lax
from jax.experimental import pallas as pl
from jax.experimental.pallas import tpu as pltpu
from jax.experimental.pallas import tpu_sc as plsc

N = 100000
D = 128
NW = 32
EPW = 3328
NP = NW * EPW
C = 256
NB = C // 128
NCHUNK = EPW // C
IPW = EPW // 128
NPB = 100096
ZPW = NPB // 16

_MESH = dict(mesh=plsc.VectorSubcoreMesh(core_axis_name="c", subcore_axis_name="s"))
_F32 = jnp.float32
_I32 = jnp.int32


def _wid():
    return lax.axis_index("c") * 16 + lax.axis_index("s")


def _lane(v16, l):
    return jnp.full((16,), v16[l], _F32)


def _bcast_dyn(ref1d, j):
    al = pl.multiple_of((j // 16) * 16, 16)
    v16 = ref1d[pl.ds(al, 16)]
    idx = jnp.full((16, 1), j - al, _I32)
    dnums = lax.GatherDimensionNumbers(
        offset_dims=(), collapsed_slice_dims=(0,), start_index_map=(0,))
    return lax.gather(v16, idx, dnums, (1,),
                      mode=lax.GatherScatterMode.PROMISE_IN_BOUNDS)


@functools.partial(
    pl.kernel,
    out_type=jax.ShapeDtypeStruct((2 * NPB,), _F32),
    scratch_types=[
        pltpu.VMEM_SHARED((NPB,), _F32),
        pltpu.VMEM((IPW, 128), _I32),
        pltpu.VMEM((128,), _F32),
        pltpu.VMEM((ZPW,), _F32),
        pltpu.SemaphoreType.DMA,
    ],
    **_MESH,
)
def _deg_kernel(row3d, out, shared, idx_v, ones_v, zbuf, sem):
    c = lax.axis_index("c")
    s = lax.axis_index("s")
    wid = c * 16 + s

    def fz(i, carry):
        zbuf[pl.ds(i * 16, 16)] = jnp.zeros((16,), _F32)
        return carry

    lax.fori_loop(0, ZPW // 16, fz, 0)
    for i in range(8):
        ones_v[pl.ds(i * 16, 16)] = jnp.ones((16,), _F32)
    pltpu.sync_copy(zbuf, shared.at[pl.ds(s * ZPW, ZPW)])
    plsc.subcore_barrier()
    pltpu.sync_copy(row3d.at[wid], idx_v)
    descs = [
        pltpu.async_copy(ones_v, shared.at[idx_v.at[b]], sem, add=True)
        for b in range(IPW)
    ]
    for d in descs:
        d.wait()
    plsc.subcore_barrier()
    pltpu.sync_copy(shared.at[pl.ds(s * ZPW, ZPW)], zbuf)
    pltpu.sync_copy(zbuf, out.at[pl.ds(c * NPB + s * ZPW, ZPW)])


def _dis_body(p_ref, dis_ref):
    deg = p_ref[0] + p_ref[1]
    dis_ref[...] = jnp.where(deg == 0.0, 0.0, lax.rsqrt(deg))


def _dis_call(partials):
    return pl.pallas_call(
        _dis_body,
        out_shape=jax.ShapeDtypeStruct((NPB // 128, 128), _F32),
    )(partials)


_BM = 512


def _mlp_body(x_ref, w1_ref, b1_ref, w2_ref, b2_ref, h_ref):
    cn = (((1,), (1,)), ((), ()))
    h1 = lax.dot_general(x_ref[...], w1_ref[...], cn, preferred_element_type=_F32)
    h1 = jnp.maximum(h1 + b1_ref[...], 0.0)
    h2 = lax.dot_general(h1, w2_ref[...], cn, preferred_element_type=_F32)
    h_ref[...] = h2 + b2_ref[...]


def _mlp_call(x_pad, W1, b1, W2, b2):
    full = pl.BlockSpec((128, 128), lambda i: (0, 0))
    brow = pl.BlockSpec((1, 128), lambda i: (0, 0))
    return pl.pallas_call(
        _mlp_body,
        grid=(NP // _BM,),
        in_specs=[pl.BlockSpec((_BM, 128), lambda i: (i, 0)), full, brow, full, brow],
        out_specs=pl.BlockSpec((_BM, 128), lambda i: (i, 0)),
        out_shape=jax.ShapeDtypeStruct((NP, 128), _F32),
    )(x_pad, W1, b1, W2, b2)


_SLICE = NP // 16
_DSL = NPB // 16


@functools.partial(
    pl.kernel,
    out_type=(
        jax.ShapeDtypeStruct((NP,), _F32),
        jax.ShapeDtypeStruct((NP, 128), _F32),
    ),
    scratch_types=[
        pltpu.VMEM((IPW * 128,), _I32),
        pltpu.VMEM((IPW * 128,), _I32),
        pltpu.VMEM((2, C), _F32),
        pltpu.VMEM((2, C), _F32),
        pltpu.VMEM((2, C), _F32),
        pltpu.VMEM((2, C, 128), _F32),
        pltpu.SemaphoreType.DMA,
        pltpu.SemaphoreType.DMA,
        pltpu.SemaphoreType.DMA,
    ],
    **_MESH,
)
def _sx1_kernel(row1d, col1d, dis1d, h, s_out, x1_out, rowf, colf,
                drv, dcv, sv, rows, sem_lin, sem_g, sem_w):
    wid = _wid()
    d1 = pltpu.async_copy(row1d.at[pl.ds(wid * EPW, EPW)], rowf, sem_lin)
    d2 = pltpu.async_copy(col1d.at[pl.ds(wid * EPW, EPW)], colf, sem_lin)
    d1.wait()
    d2.wait()

    def fire(t, p):
        cidx = colf.at[pl.ds(t * C, C)]
        descs = [pltpu.async_copy(h.at[cidx], rows.at[p], sem_g)]
        for b in range(NB):
            sl = pl.ds(b * 128, 128)
            rb = rowf.at[pl.ds(t * C + b * 128, 128)]
            cb = colf.at[pl.ds(t * C + b * 128, 128)]
            descs.append(pltpu.async_copy(dis1d.at[rb], drv.at[p, sl], sem_g))
            descs.append(pltpu.async_copy(dis1d.at[cb], dcv.at[p, sl], sem_g))
        return descs

    dcur = fire(0, 0)
    wr = {0: [], 1: []}
    for t in range(NCHUNK):
        p = t % 2
        q = 1 - p
        for d in wr[q]:
            d.wait()
        wr[q] = []
        dnext = fire(t + 1, q) if t + 1 < NCHUNK else []
        for d in dcur:
            d.wait()
        for i in range(C // 16):
            sl = pl.ds(i * 16, 16)
            sv[p, sl] = drv[p, sl] * dcv[p, sl]

        @plsc.parallel_loop(0, C, 1, unroll=4)
        def rowfn(j, p=p):
            sj = _bcast_dyn(sv.at[p], j)
            for v in range(8):
                sl = pl.ds(v * 16, 16)
                rows[p, j, sl] = jnp.maximum(rows[p, j, sl] * sj, 0.0)
        base = wid * EPW + t * C
        wr[p] = [
            pltpu.async_copy(sv.at[p], s_out.at[pl.ds(base, C)], sem_w),
            pltpu.async_copy(rows.at[p], x1_out.at[pl.ds(base, C)], sem_w),
        ]
        dcur = dnext
    for p in (0, 1):
        for d in wr[p]:
            d.wait()


@functools.partial(
    pl.kernel,
    out_type=jax.ShapeDtypeStruct((NP, 128), _F32),
    scratch_types=[
        pltpu.VMEM((16,), _F32),
        pltpu.VMEM((IPW * 128,), _I32),
        pltpu.VMEM((C,), _I32),
        pltpu.VMEM((C,), _I32),
        pltpu.VMEM((C,), _F32),
        pltpu.VMEM((2, C), _F32),
        pltpu.VMEM((C,), _F32),
        pltpu.VMEM((C, 128), _F32),
        pltpu.VMEM((2, C, 128), _F32),
        pltpu.SemaphoreType.DMA,
        pltpu.SemaphoreType.DMA,
        pltpu.SemaphoreType.DMA,
    ],
    **_MESH,
)
def _chain_kernel(col1d, s1d, h, x1, g16, out, gbuf, colf, mA, mB, qv, smv,
                  wv, acc, rows, sem_lin, sem_rows, sem_idx):
    wid = _wid()
    pltpu.sync_copy(g16, gbuf)
    pltpu.sync_copy(col1d.at[pl.ds(wid * EPW, EPW)], colf)

    def chunk(t, carry):
        base = wid * EPW + t * C
        esl = pl.ds(base, C)
        idx0 = colf.at[pl.ds(t * C, C)]
        d_rows = [pltpu.async_copy(x1.at[idx0], rows.at[1], sem_rows)]
        d_sm, d_m = [], []
        for b in range(NB):
            sl = pl.ds(b * 128, 128)
            ib = colf.at[pl.ds(t * C + b * 128, 128)]
            d_sm.append(pltpu.async_copy(s1d.at[ib], smv.at[0, sl], sem_idx))
            d_m.append(
                pltpu.async_copy(col1d.at[ib], mB.at[sl], sem_idx))
        dh = pltpu.async_copy(h.at[esl], acc, sem_lin)
        dx = pltpu.async_copy(x1.at[esl], rows.at[0], sem_lin)
        dq = pltpu.async_copy(s1d.at[esl], qv, sem_lin)
        dh.wait()
        dx.wait()
        dq.wait()
        gv = gbuf[pl.ds(0, 16)]
        g0 = _lane(gv, 0)
        g1 = _lane(gv, 1)

        @plsc.parallel_loop(0, C, 1, unroll=4)
        def initrow(j):
            for v in range(8):
                sl = pl.ds(v * 16, 16)
                acc[j, sl] = acc[j, sl] * g0 + rows[0, j, sl] * g1

        m_cur, m_nxt = mB, mA
        pc, ps = 1, 0
        for k in range(2, 11):
            for d in d_sm:
                d.wait()
            for d in d_m:
                d.wait()
            gk = _lane(gv, k)
            for i in range(C // 16):
                sl = pl.ds(i * 16, 16)
                wv[sl] = qv[sl] * gk
                if k < 10:
                    qv[sl] = qv[sl] * smv[ps, sl]
            d_sm, d_m, d_next = [], [], []
            if k < 10:
                d_next.append(
                    pltpu.async_copy(x1.at[m_cur], rows.at[1 - pc], sem_rows))
                if k < 9:
                    for b in range(NB):
                        sl = pl.ds(b * 128, 128)
                        mb = m_cur.at[sl]
                        d_sm.append(
                            pltpu.async_copy(s1d.at[mb], smv.at[1 - ps, sl],
                                             sem_idx))
                        d_m.append(
                            pltpu.async_copy(col1d.at[mb], m_nxt.at[sl],
                                             sem_idx))
            for d in d_rows:
                d.wait()

            @plsc.parallel_loop(0, C, 1, unroll=8)
            def accrow(j, pc=pc):
                wj = _bcast_dyn(wv, j)
                for v in range(8):
                    sl = pl.ds(v * 16, 16)
                    acc[j, sl] = acc[j, sl] + rows[pc, j, sl] * wj
            d_rows = d_next
            pc = 1 - pc
            ps = 1 - ps
            m_cur, m_nxt = m_nxt, m_cur
        pltpu.sync_copy(acc, out.at[esl])
        return carry

    lax.fori_loop(0, NCHUNK, chunk, 0)


def kernel(x, edge_index, W1, b1, W2, b2, gamma):
    ei = edge_index.astype(_I32)
    row = ei[0]
    col = ei[1]
    pad = NP - N
    pad_iota = jnp.arange(pad, dtype=_I32)
    row_pad = jnp.concatenate([row, N + pad_iota % (NPB - N)])
    col_pad = jnp.concatenate([col, (pad_iota * 15) % N])
    row3d = row_pad.reshape(NW, IPW, 128)
    col3d = col_pad.reshape(NW, IPW, 128)
    x_pad = jnp.pad(x, ((0, NP - N), (0, 0)))
    g16 = jnp.pad(gamma.astype(_F32), (0, 16 - gamma.shape[0]))

    partials = _deg_kernel(row3d)
    dis = _dis_call(partials.reshape(2, NPB // 128, 128)).reshape(NPB)
    h = _mlp_call(x_pad, W1, b1.reshape(1, D), W2, b2.reshape(1, D))
    s, x1 = _sx1_kernel(row_pad, col_pad, dis, h)
    out = _chain_kernel(col_pad, s, h, x1, g16)
    return out[:N]


# speedup vs baseline: 1.1255x; 1.1255x over previous
import functools

import jax
import jax.numpy as jnp
from jax import lax
from jax.experimental import pallas as pl
from jax.experimental.pallas import tpu as pltpu
from jax.experimental.pallas import tpu_sc as plsc

N = 100000
D = 128
NW = 32
EPW = 3328
NP = NW * EPW
C = 256
NB = C // 128
NCHUNK = EPW // C
IPW = EPW // 128
NPB = 100352
ZPW = NPB // 16

_MESH = dict(mesh=plsc.VectorSubcoreMesh(core_axis_name="c", subcore_axis_name="s"))
_F32 = jnp.float32
_I32 = jnp.int32


def _wid():
    return lax.axis_index("c") * 16 + lax.axis_index("s")


def _lane(v16, l):
    return jnp.full((16,), v16[l], _F32)


def _bcast_dyn(ref1d, j):
    al = pl.multiple_of((j // 16) * 16, 16)
    v16 = ref1d[pl.ds(al, 16)]
    idx = jnp.full((16, 1), j - al, _I32)
    dnums = lax.GatherDimensionNumbers(
        offset_dims=(), collapsed_slice_dims=(0,), start_index_map=(0,))
    return lax.gather(v16, idx, dnums, (1,),
                      mode=lax.GatherScatterMode.PROMISE_IN_BOUNDS)


@functools.partial(
    pl.kernel,
    out_type=jax.ShapeDtypeStruct((2 * NPB,), _F32),
    scratch_types=[
        pltpu.VMEM_SHARED((NPB,), _F32),
        pltpu.VMEM((IPW, 128), _I32),
        pltpu.VMEM((128,), _F32),
        pltpu.VMEM((ZPW,), _F32),
        pltpu.SemaphoreType.DMA,
    ],
    **_MESH,
)
def _deg_kernel(row3d, out, shared, idx_v, ones_v, zbuf, sem):
    c = lax.axis_index("c")
    s = lax.axis_index("s")
    wid = c * 16 + s

    def fz(i, carry):
        zbuf[pl.ds(i * 16, 16)] = jnp.zeros((16,), _F32)
        return carry

    lax.fori_loop(0, ZPW // 16, fz, 0)
    for i in range(8):
        ones_v[pl.ds(i * 16, 16)] = jnp.ones((16,), _F32)
    pltpu.sync_copy(zbuf, shared.at[pl.ds(s * ZPW, ZPW)])
    plsc.subcore_barrier()
    pltpu.sync_copy(row3d.at[wid], idx_v)
    descs = [
        pltpu.async_copy(ones_v, shared.at[idx_v.at[b]], sem, add=True)
        for b in range(IPW)
    ]
    for d in descs:
        d.wait()
    plsc.subcore_barrier()
    pltpu.sync_copy(shared.at[pl.ds(s * ZPW, ZPW)], zbuf)
    pltpu.sync_copy(zbuf, out.at[pl.ds(c * NPB + s * ZPW, ZPW)])


_BM = 512
_NDB = NPB // (8 * 128)


def _mlp_body(x_ref, w1_ref, b1_ref, w2_ref, b2_ref, p_ref, h_ref, dis_ref):
    cn = (((1,), (1,)), ((), ()))
    h1 = lax.dot_general(x_ref[...], w1_ref[...], cn, preferred_element_type=_F32)
    h1 = jnp.maximum(h1 + b1_ref[...], 0.0)
    h2 = lax.dot_general(h1, w2_ref[...], cn, preferred_element_type=_F32)
    h_ref[...] = h2 + b2_ref[...]
    deg = p_ref[0] + p_ref[1]
    dis_ref[...] = jnp.where(deg == 0.0, 0.0, lax.rsqrt(deg))


def _mlp_call(x_pad, W1, b1, W2, b2, partials):
    full = pl.BlockSpec((128, 128), lambda i: (0, 0))
    brow = pl.BlockSpec((1, 128), lambda i: (0, 0))
    dmap = lambda i: (jnp.minimum(i, _NDB - 1), 0)
    h, dis = pl.pallas_call(
        _mlp_body,
        grid=(NP // _BM,),
        in_specs=[pl.BlockSpec((_BM, 128), lambda i: (i, 0)), full, brow, full,
                  brow,
                  pl.BlockSpec((2, 8, 128),
                               lambda i: (0, jnp.minimum(i, _NDB - 1), 0))],
        out_specs=[pl.BlockSpec((_BM, 128), lambda i: (i, 0)),
                   pl.BlockSpec((8, 128), dmap)],
        out_shape=[jax.ShapeDtypeStruct((NP, 128), _F32),
                   jax.ShapeDtypeStruct((NPB // 128, 128), _F32)],
    )(x_pad, W1, b1, W2, b2, partials)
    return h, dis


_SLICE = NP // 16
_DSL = NPB // 16


@functools.partial(
    pl.kernel,
    out_type=(
        jax.ShapeDtypeStruct((NP,), _F32),
        jax.ShapeDtypeStruct((NP, 128), _F32),
    ),
    scratch_types=[
        pltpu.VMEM((IPW * 128,), _I32),
        pltpu.VMEM((IPW * 128,), _I32),
        pltpu.VMEM((2, C), _F32),
        pltpu.VMEM((2, C), _F32),
        pltpu.VMEM((2, C), _F32),
        pltpu.VMEM((2, C, 128), _F32),
        pltpu.SemaphoreType.DMA,
        pltpu.SemaphoreType.DMA,
        pltpu.SemaphoreType.DMA,
    ],
    **_MESH,
)
def _sx1_kernel(row1d, col1d, dis1d, h, s_out, x1_out, rowf, colf,
                drv, dcv, sv, rows, sem_lin, sem_g, sem_w):
    wid = _wid()
    d1 = pltpu.async_copy(row1d.at[pl.ds(wid * EPW, EPW)], rowf, sem_lin)
    d2 = pltpu.async_copy(col1d.at[pl.ds(wid * EPW, EPW)], colf, sem_lin)
    d1.wait()
    d2.wait()

    def fire(t, p):
        cidx = colf.at[pl.ds(t * C, C)]
        descs = [pltpu.async_copy(h.at[cidx], rows.at[p], sem_g)]
        for b in range(NB):
            sl = pl.ds(b * 128, 128)
            rb = rowf.at[pl.ds(t * C + b * 128, 128)]
            cb = colf.at[pl.ds(t * C + b * 128, 128)]
            descs.append(pltpu.async_copy(dis1d.at[rb], drv.at[p, sl], sem_g))
            descs.append(pltpu.async_copy(dis1d.at[cb], dcv.at[p, sl], sem_g))
        return descs

    dcur = fire(0, 0)
    wr = {0: [], 1: []}
    for t in range(NCHUNK):
        p = t % 2
        q = 1 - p
        for d in wr[q]:
            d.wait()
        wr[q] = []
        dnext = fire(t + 1, q) if t + 1 < NCHUNK else []
        for d in dcur:
            d.wait()
        for i in range(C // 16):
            sl = pl.ds(i * 16, 16)
            sv[p, sl] = drv[p, sl] * dcv[p, sl]

        @plsc.parallel_loop(0, C, 1, unroll=4)
        def rowfn(j, p=p):
            sj = _bcast_dyn(sv.at[p], j)
            for v in range(8):
                sl = pl.ds(v * 16, 16)
                rows[p, j, sl] = jnp.maximum(rows[p, j, sl] * sj, 0.0)
        base = wid * EPW + t * C
        wr[p] = [
            pltpu.async_copy(sv.at[p], s_out.at[pl.ds(base, C)], sem_w),
            pltpu.async_copy(rows.at[p], x1_out.at[pl.ds(base, C)], sem_w),
        ]
        dcur = dnext
    for p in (0, 1):
        for d in wr[p]:
            d.wait()


@functools.partial(
    pl.kernel,
    out_type=jax.ShapeDtypeStruct((NP, 128), _F32),
    scratch_types=[
        pltpu.VMEM((16,), _F32),
        pltpu.VMEM((IPW * 128,), _I32),
        pltpu.VMEM((C,), _I32),
        pltpu.VMEM((C,), _I32),
        pltpu.VMEM((C,), _F32),
        pltpu.VMEM((2, C), _F32),
        pltpu.VMEM((C,), _F32),
        pltpu.VMEM((C, 128), _F32),
        pltpu.VMEM((2, C, 128), _F32),
        pltpu.SemaphoreType.DMA,
        pltpu.SemaphoreType.DMA,
        pltpu.SemaphoreType.DMA,
    ],
    **_MESH,
)
def _chain_kernel(col1d, s1d, h, x1, g16, out, gbuf, colf, mA, mB, qv, smv,
                  wv, acc, rows, sem_lin, sem_rows, sem_idx):
    wid = _wid()
    pltpu.sync_copy(g16, gbuf)
    pltpu.sync_copy(col1d.at[pl.ds(wid * EPW, EPW)], colf)

    def chunk(t, carry):
        base = wid * EPW + t * C
        esl = pl.ds(base, C)
        idx0 = colf.at[pl.ds(t * C, C)]
        d_rows = [pltpu.async_copy(x1.at[idx0], rows.at[1], sem_rows)]
        d_sm, d_m = [], []
        for b in range(NB):
            sl = pl.ds(b * 128, 128)
            ib = colf.at[pl.ds(t * C + b * 128, 128)]
            d_sm.append(pltpu.async_copy(s1d.at[ib], smv.at[0, sl], sem_idx))
            d_m.append(
                pltpu.async_copy(col1d.at[ib], mB.at[sl], sem_idx))
        dh = pltpu.async_copy(h.at[esl], acc, sem_lin)
        dx = pltpu.async_copy(x1.at[esl], rows.at[0], sem_lin)
        dq = pltpu.async_copy(s1d.at[esl], qv, sem_lin)
        dh.wait()
        dx.wait()
        dq.wait()
        gv = gbuf[pl.ds(0, 16)]
        g0 = _lane(gv, 0)
        g1 = _lane(gv, 1)

        @plsc.parallel_loop(0, C, 1, unroll=4)
        def initrow(j):
            for v in range(8):
                sl = pl.ds(v * 16, 16)
                acc[j, sl] = acc[j, sl] * g0 + rows[0, j, sl] * g1

        m_cur, m_nxt = mB, mA
        pc, ps = 1, 0
        for k in range(2, 11):
            for d in d_sm:
                d.wait()
            for d in d_m:
                d.wait()
            gk = _lane(gv, k)
            for i in range(C // 16):
                sl = pl.ds(i * 16, 16)
                wv[sl] = qv[sl] * gk
                if k < 10:
                    qv[sl] = qv[sl] * smv[ps, sl]
            d_sm, d_m, d_next = [], [], []
            if k < 10:
                d_next.append(
                    pltpu.async_copy(x1.at[m_cur], rows.at[1 - pc], sem_rows))
                if k < 9:
                    for b in range(NB):
                        sl = pl.ds(b * 128, 128)
                        mb = m_cur.at[sl]
                        d_sm.append(
                            pltpu.async_copy(s1d.at[mb], smv.at[1 - ps, sl],
                                             sem_idx))
                        d_m.append(
                            pltpu.async_copy(col1d.at[mb], m_nxt.at[sl],
                                             sem_idx))
            for d in d_rows:
                d.wait()

            @plsc.parallel_loop(0, C, 1, unroll=4)
            def accrow(j, pc=pc):
                wj = _bcast_dyn(wv, j)
                for v in range(8):
                    sl = pl.ds(v * 16, 16)
                    acc[j, sl] = acc[j, sl] + rows[pc, j, sl] * wj
            d_rows = d_next
            pc = 1 - pc
            ps = 1 - ps
            m_cur, m_nxt = m_nxt, m_cur
        pltpu.sync_copy(acc, out.at[esl])
        return carry

    lax.fori_loop(0, NCHUNK, chunk, 0)


def kernel(x, edge_index, W1, b1, W2, b2, gamma):
    ei = edge_index.astype(_I32)
    row = ei[0]
    col = ei[1]
    pad = NP - N
    pad_iota = jnp.arange(pad, dtype=_I32)
    row_pad = jnp.concatenate([row, N + pad_iota % (NPB - N)])
    col_pad = jnp.concatenate([col, (pad_iota * 15) % N])
    row3d = row_pad.reshape(NW, IPW, 128)
    col3d = col_pad.reshape(NW, IPW, 128)
    x_pad = jnp.pad(x, ((0, NP - N), (0, 0)))
    g16 = jnp.pad(gamma.astype(_F32), (0, 16 - gamma.shape[0]))

    partials = _deg_kernel(row3d)
    h, dis = _mlp_call(x_pad, W1, b1.reshape(1, D), W2, b2.reshape(1, D),
                       partials.reshape(2, NPB // 128, 128))
    dis = dis.reshape(NPB)
    s, x1 = _sx1_kernel(row_pad, col_pad, dis, h)
    out = _chain_kernel(col_pad, s, h, x1, g16)
    return out[:N]


# speedup vs baseline: 1.1255x; 1.0000x over previous
import functools

import jax
import jax.numpy as jnp
from jax import lax
from jax.experimental import pallas as pl
from jax.experimental.pallas import tpu as pltpu
from jax.experimental.pallas import tpu_sc as plsc

N = 100000
D = 128
NW = 32
EPW = 3328
NP = NW * EPW
C = 256
NB = C // 128
NCHUNK = EPW // C
IPW = EPW // 128
NPB = 100352
ZPW = NPB // 16

_MESH = dict(mesh=plsc.VectorSubcoreMesh(core_axis_name="c", subcore_axis_name="s"))
_F32 = jnp.float32
_I32 = jnp.int32


def _wid():
    return lax.axis_index("c") * 16 + lax.axis_index("s")


def _lane(v16, l):
    return jnp.full((16,), v16[l], _F32)


def _bcast_dyn(ref1d, j):
    al = pl.multiple_of((j // 16) * 16, 16)
    v16 = ref1d[pl.ds(al, 16)]
    idx = jnp.full((16, 1), j - al, _I32)
    dnums = lax.GatherDimensionNumbers(
        offset_dims=(), collapsed_slice_dims=(0,), start_index_map=(0,))
    return lax.gather(v16, idx, dnums, (1,),
                      mode=lax.GatherScatterMode.PROMISE_IN_BOUNDS)


@functools.partial(
    pl.kernel,
    out_type=jax.ShapeDtypeStruct((2 * NPB,), _F32),
    scratch_types=[
        pltpu.VMEM_SHARED((NPB,), _F32),
        pltpu.VMEM((IPW, 128), _I32),
        pltpu.VMEM((128,), _F32),
        pltpu.VMEM((ZPW,), _F32),
        pltpu.SemaphoreType.DMA,
    ],
    **_MESH,
)
def _deg_kernel(row3d, out, shared, idx_v, ones_v, zbuf, sem):
    c = lax.axis_index("c")
    s = lax.axis_index("s")
    wid = c * 16 + s

    def fz(i, carry):
        zbuf[pl.ds(i * 16, 16)] = jnp.zeros((16,), _F32)
        return carry

    lax.fori_loop(0, ZPW // 16, fz, 0)
    for i in range(8):
        ones_v[pl.ds(i * 16, 16)] = jnp.ones((16,), _F32)
    pltpu.sync_copy(zbuf, shared.at[pl.ds(s * ZPW, ZPW)])
    plsc.subcore_barrier()
    pltpu.sync_copy(row3d.at[wid], idx_v)
    descs = [
        pltpu.async_copy(ones_v, shared.at[idx_v.at[b]], sem, add=True)
        for b in range(IPW)
    ]
    for d in descs:
        d.wait()
    plsc.subcore_barrier()
    pltpu.sync_copy(shared.at[pl.ds(s * ZPW, ZPW)], zbuf)
    pltpu.sync_copy(zbuf, out.at[pl.ds(c * NPB + s * ZPW, ZPW)])


_BM = 512
_NDB = NPB // (8 * 128)


def _mlp_body(x_ref, w1_ref, b1_ref, w2_ref, b2_ref, p_ref, h_ref, dis_ref):
    cn = (((1,), (1,)), ((), ()))
    h1 = lax.dot_general(x_ref[...], w1_ref[...], cn, preferred_element_type=_F32)
    h1 = jnp.maximum(h1 + b1_ref[...], 0.0)
    h2 = lax.dot_general(h1, w2_ref[...], cn, preferred_element_type=_F32)
    h_ref[...] = h2 + b2_ref[...]
    deg = p_ref[0] + p_ref[1]
    dis_ref[...] = jnp.where(deg == 0.0, 0.0, lax.rsqrt(deg))


def _mlp_call(x_pad, W1, b1, W2, b2, partials):
    full = pl.BlockSpec((128, 128), lambda i: (0, 0))
    brow = pl.BlockSpec((1, 128), lambda i: (0, 0))
    dmap = lambda i: (jnp.minimum(i, _NDB - 1), 0)
    h, dis = pl.pallas_call(
        _mlp_body,
        grid=(NP // _BM,),
        in_specs=[pl.BlockSpec((_BM, 128), lambda i: (i, 0)), full, brow, full,
                  brow,
                  pl.BlockSpec((2, 8, 128),
                               lambda i: (0, jnp.minimum(i, _NDB - 1), 0))],
        out_specs=[pl.BlockSpec((_BM, 128), lambda i: (i, 0)),
                   pl.BlockSpec((8, 128), dmap)],
        out_shape=[jax.ShapeDtypeStruct((NP, 128), _F32),
                   jax.ShapeDtypeStruct((NPB // 128, 128), _F32)],
    )(x_pad, W1, b1, W2, b2, partials)
    return h, dis


_SLICE = NP // 16
_DSL = NPB // 16


@functools.partial(
    pl.kernel,
    out_type=(
        jax.ShapeDtypeStruct((NP,), _F32),
        jax.ShapeDtypeStruct((NP, 128), _F32),
    ),
    scratch_types=[
        pltpu.VMEM((IPW * 128,), _I32),
        pltpu.VMEM((IPW * 128,), _I32),
        pltpu.VMEM((2, C), _F32),
        pltpu.VMEM((2, C), _F32),
        pltpu.VMEM((2, C), _F32),
        pltpu.VMEM((2, C, 128), _F32),
        pltpu.SemaphoreType.DMA,
        pltpu.SemaphoreType.DMA,
        pltpu.SemaphoreType.DMA,
    ],
    **_MESH,
)
def _sx1_kernel(row1d, col1d, dis1d, h, s_out, x1_out, rowf, colf,
                drv, dcv, sv, rows, sem_lin, sem_g, sem_w):
    wid = _wid()
    d1 = pltpu.async_copy(row1d.at[pl.ds(wid * EPW, EPW)], rowf, sem_lin)
    d2 = pltpu.async_copy(col1d.at[pl.ds(wid * EPW, EPW)], colf, sem_lin)
    d1.wait()
    d2.wait()

    def fire(t, p):
        descs = [
            pltpu.async_copy(h.at[colf.at[pl.ds(t * C + b * 128, 128)]],
                             rows.at[p, pl.ds(b * 128, 128)], sem_g)
            for b in range(NB)
        ]
        for b in range(NB):
            sl = pl.ds(b * 128, 128)
            rb = rowf.at[pl.ds(t * C + b * 128, 128)]
            cb = colf.at[pl.ds(t * C + b * 128, 128)]
            descs.append(pltpu.async_copy(dis1d.at[rb], drv.at[p, sl], sem_g))
            descs.append(pltpu.async_copy(dis1d.at[cb], dcv.at[p, sl], sem_g))
        return descs

    dcur = fire(0, 0)
    wr = {0: [], 1: []}
    for t in range(NCHUNK):
        p = t % 2
        q = 1 - p
        for d in wr[q]:
            d.wait()
        wr[q] = []
        dnext = fire(t + 1, q) if t + 1 < NCHUNK else []
        for d in dcur:
            d.wait()
        for i in range(C // 16):
            sl = pl.ds(i * 16, 16)
            sv[p, sl] = drv[p, sl] * dcv[p, sl]

        @plsc.parallel_loop(0, C, 1, unroll=4)
        def rowfn(j, p=p):
            sj = _bcast_dyn(sv.at[p], j)
            for v in range(8):
                sl = pl.ds(v * 16, 16)
                rows[p, j, sl] = jnp.maximum(rows[p, j, sl] * sj, 0.0)
        base = wid * EPW + t * C
        wr[p] = [
            pltpu.async_copy(sv.at[p], s_out.at[pl.ds(base, C)], sem_w),
            pltpu.async_copy(rows.at[p], x1_out.at[pl.ds(base, C)], sem_w),
        ]
        dcur = dnext
    for p in (0, 1):
        for d in wr[p]:
            d.wait()


@functools.partial(
    pl.kernel,
    out_type=jax.ShapeDtypeStruct((NP, 128), _F32),
    scratch_types=[
        pltpu.VMEM((16,), _F32),
        pltpu.VMEM((IPW * 128,), _I32),
        pltpu.VMEM((C,), _I32),
        pltpu.VMEM((C,), _I32),
        pltpu.VMEM((C,), _F32),
        pltpu.VMEM((2, C), _F32),
        pltpu.VMEM((C,), _F32),
        pltpu.VMEM((C, 128), _F32),
        pltpu.VMEM((2, C, 128), _F32),
        pltpu.SemaphoreType.DMA,
        pltpu.SemaphoreType.DMA,
        pltpu.SemaphoreType.DMA,
    ],
    **_MESH,
)
def _chain_kernel(col1d, s1d, h, x1, g16, out, gbuf, colf, mA, mB, qv, smv,
                  wv, acc, rows, sem_lin, sem_rows, sem_idx):
    wid = _wid()
    pltpu.sync_copy(g16, gbuf)
    pltpu.sync_copy(col1d.at[pl.ds(wid * EPW, EPW)], colf)

    def chunk(t, carry):
        base = wid * EPW + t * C
        esl = pl.ds(base, C)
        d_rows = [
            pltpu.async_copy(x1.at[colf.at[pl.ds(t * C + b * 128, 128)]],
                             rows.at[1, pl.ds(b * 128, 128)], sem_rows)
            for b in range(NB)
        ]
        d_sm, d_m = [], []
        for b in range(NB):
            sl = pl.ds(b * 128, 128)
            ib = colf.at[pl.ds(t * C + b * 128, 128)]
            d_sm.append(pltpu.async_copy(s1d.at[ib], smv.at[0, sl], sem_idx))
            d_m.append(
                pltpu.async_copy(col1d.at[ib], mB.at[sl], sem_idx))
        dh = pltpu.async_copy(h.at[esl], acc, sem_lin)
        dx = pltpu.async_copy(x1.at[esl], rows.at[0], sem_lin)
        dq = pltpu.async_copy(s1d.at[esl], qv, sem_lin)
        dh.wait()
        dx.wait()
        dq.wait()
        gv = gbuf[pl.ds(0, 16)]
        g0 = _lane(gv, 0)
        g1 = _lane(gv, 1)

        @plsc.parallel_loop(0, C, 1, unroll=4)
        def initrow(j):
            for v in range(8):
                sl = pl.ds(v * 16, 16)
                acc[j, sl] = acc[j, sl] * g0 + rows[0, j, sl] * g1

        m_cur, m_nxt = mB, mA
        pc, ps = 1, 0
        for k in range(2, 11):
            for d in d_sm:
                d.wait()
            for d in d_m:
                d.wait()
            gk = _lane(gv, k)
            for i in range(C // 16):
                sl = pl.ds(i * 16, 16)
                wv[sl] = qv[sl] * gk
                if k < 10:
                    qv[sl] = qv[sl] * smv[ps, sl]
            d_sm, d_m, d_next = [], [], []
            if k < 10:
                for b in range(NB):
                    sl = pl.ds(b * 128, 128)
                    d_next.append(
                        pltpu.async_copy(x1.at[m_cur.at[sl]],
                                         rows.at[1 - pc, sl], sem_rows))
                if k < 9:
                    for b in range(NB):
                        sl = pl.ds(b * 128, 128)
                        mb = m_cur.at[sl]
                        d_sm.append(
                            pltpu.async_copy(s1d.at[mb], smv.at[1 - ps, sl],
                                             sem_idx))
                        d_m.append(
                            pltpu.async_copy(col1d.at[mb], m_nxt.at[sl],
                                             sem_idx))
            for d in d_rows:
                d.wait()

            @plsc.parallel_loop(0, C, 1, unroll=4)
            def accrow(j, pc=pc):
                wj = _bcast_dyn(wv, j)
                for v in range(8):
                    sl = pl.ds(v * 16, 16)
                    acc[j, sl] = acc[j, sl] + rows[pc, j, sl] * wj
            d_rows = d_next
            pc = 1 - pc
            ps = 1 - ps
            m_cur, m_nxt = m_nxt, m_cur
        pltpu.sync_copy(acc, out.at[esl])
        return carry

    lax.fori_loop(0, NCHUNK, chunk, 0)


def kernel(x, edge_index, W1, b1, W2, b2, gamma):
    ei = edge_index.astype(_I32)
    row = ei[0]
    col = ei[1]
    pad = NP - N
    pad_iota = jnp.arange(pad, dtype=_I32)
    row_pad = jnp.concatenate([row, N + pad_iota % (NPB - N)])
    col_pad = jnp.concatenate([col, (pad_iota * 15) % N])
    row3d = row_pad.reshape(NW, IPW, 128)
    col3d = col_pad.reshape(NW, IPW, 128)
    x_pad = jnp.pad(x, ((0, NP - N), (0, 0)))
    g16 = jnp.pad(gamma.astype(_F32), (0, 16 - gamma.shape[0]))

    partials = _deg_kernel(row3d)
    h, dis = _mlp_call(x_pad, W1, b1.reshape(1, D), W2, b2.reshape(1, D),
                       partials.reshape(2, NPB // 128, 128))
    dis = dis.reshape(NPB)
    s, x1 = _sx1_kernel(row_pad, col_pad, dis, h)
    out = _chain_kernel(col_pad, s, h, x1, g16)
    return out[:N]
